# baseline (speedup 1.0000x reference)
import numpy as np
import jax
import jax.numpy as jnp
from jax import lax
from jax.experimental import pallas as pl

_LOG_K = float(np.log(64))


def _body(u_ref, vf_ref, bf_ref, ts_ref, coef_ref, vp_ref, lnvt_ref, lv0_ref):
    R = u_ref.shape[0]
    u = u_ref[...]
    vf = vf_ref[...]
    bf = bf_ref[...]
    ts = ts_ref[...]
    coef = coef_ref[...]

    z = u[0:1, 0:1] * 0.0
    neg30 = jnp.log(z + 1e-30)
    pos = jnp.log(z + 1.0)

    iota_t = lax.broadcasted_iota(jnp.int32, (64, 128), 1).astype(jnp.float32)
    tsoh = (iota_t == ts).astype(jnp.float32)
    lal1_b = jnp.dot(tsoh, coef, precision=lax.Precision.HIGHEST,
                     preferred_element_type=jnp.float32)
    la_b = lal1_b[:, 0:1]
    b_b = lal1_b[:, 1:2] - _LOG_K

    a_hit = pos + la_b
    m_h = jnp.maximum(a_hit, b_b)
    q_hit_b = m_h + jnp.log(jnp.exp(a_hit - m_h) + jnp.exp(b_b - m_h))
    a_miss = neg30 + la_b
    m_m = jnp.maximum(a_miss, b_b)
    q_miss_b = m_m + jnp.log(jnp.exp(a_miss - m_m) + jnp.exp(b_b - m_m))
    q_b = jnp.concatenate([q_hit_b, q_miss_b], axis=1)

    iota_c = lax.broadcasted_iota(jnp.int32, (R, 64), 1).astype(jnp.float32)
    boh = (iota_c == bf).astype(jnp.float32)
    q = jnp.dot(boh, q_b, precision=lax.Precision.HIGHEST,
                preferred_element_type=jnp.float32)
    q_hit = q[:, 0:1]
    q_miss = q[:, 1:2]

    oh_v = iota_c == vf

    u_v = jnp.max(jnp.where(oh_v, u, -1.0), axis=1, keepdims=True)
    u_top = jnp.max(jnp.where(oh_v, -1.0, u), axis=1, keepdims=True)
    c1 = jnp.min(jnp.where((u == u_top) & (~oh_v), iota_c, 64.0),
                 axis=1, keepdims=True)

    g_v = -jnp.log(-jnp.log(u_v + 1e-30) + 1e-30)
    g_1 = -jnp.log(-jnp.log(u_top + 1e-30) + 1e-30)
    s_v = g_v + q_hit
    s_1 = g_1 + q_miss

    widx = jnp.where(s_v > s_1, vf,
                     jnp.where(s_1 > s_v, c1, jnp.minimum(vf, c1)))

    vp = (iota_c == widx).astype(jnp.float32)
    vp_ref[...] = vp
    lnvt_ref[...] = jnp.where(iota_c == widx, pos, neg30)
    lv0_ref[...] = jnp.where(oh_v, pos, neg30)


def kernel(v, time_step, batch, u, log_alphas_bar, log_1_min_alphas_bar):
    N, C = u.shape
    R = 1024
    G = N // R
    vf = v.astype(jnp.float32).reshape(N, 1)
    bf = batch.astype(jnp.float32).reshape(N, 1)
    tsf = time_step.astype(jnp.float32).reshape(-1, 1)
    T = log_alphas_bar.shape[0]
    coef = jnp.zeros((128, 2), jnp.float32)
    coef = coef.at[:T, 0].set(log_alphas_bar).at[:T, 1].set(log_1_min_alphas_bar)

    row_spec = pl.BlockSpec((R, C), lambda i: (i, 0))
    col_spec = pl.BlockSpec((R, 1), lambda i: (i, 0))
    ts_spec = pl.BlockSpec((64, 1), lambda i: (0, 0))
    coef_spec = pl.BlockSpec((128, 2), lambda i: (0, 0))
    out_sds = jax.ShapeDtypeStruct((N, C), jnp.float32)

    vp, lnvt, lv0 = pl.pallas_call(
        _body,
        grid=(G,),
        in_specs=[row_spec, col_spec, col_spec, ts_spec, coef_spec],
        out_specs=[row_spec, row_spec, row_spec],
        out_shape=[out_sds, out_sds, out_sds],
    )(u, vf, bf, tsf, coef)
    return (vp, lnvt, lv0)


# speedup vs baseline: 1.1011x; 1.1011x over previous
import numpy as np
import jax
import jax.numpy as jnp
from jax import lax
from jax.experimental import pallas as pl

_LOG_K = float(np.log(64))


def _body(u_ref, vi_ref, bi_ref, ts_ref, coef_ref, vp_ref, lnvt_ref, lv0_ref):
    R = u_ref.shape[0]
    u = u_ref[...]
    vi = vi_ref[...]
    bi = bi_ref[...]
    ts = ts_ref[...]
    coef = coef_ref[...]

    z = u[0:1, 0:1] * 0.0
    neg30 = jnp.log(z + 1e-30)
    pos = jnp.log(z + 1.0)

    iota_t = lax.broadcasted_iota(jnp.int32, (64, 128), 1)
    tsoh = (iota_t == ts).astype(jnp.float32)
    lal1_b = jnp.dot(tsoh, coef, precision=lax.Precision.HIGHEST,
                     preferred_element_type=jnp.float32)
    la_b = lal1_b[:, 0:1]
    b_b = lal1_b[:, 1:2] - _LOG_K

    a_hit = pos + la_b
    m_h = jnp.maximum(a_hit, b_b)
    q_hit_b = m_h + jnp.log(jnp.exp(a_hit - m_h) + jnp.exp(b_b - m_h))
    a_miss = neg30 + la_b
    m_m = jnp.maximum(a_miss, b_b)
    q_miss_b = m_m + jnp.log(jnp.exp(a_miss - m_m) + jnp.exp(b_b - m_m))
    q_b = jnp.concatenate([q_hit_b, q_miss_b], axis=1)

    iota_c = lax.broadcasted_iota(jnp.int32, (R, 64), 1)
    oh_v = iota_c == vi
    oh_vf = oh_v.astype(jnp.float32)
    boh = (iota_c == bi).astype(jnp.float32)

    q = jnp.dot(boh, q_b, precision=lax.Precision.HIGHEST,
                preferred_element_type=jnp.float32)
    q_hit = q[:, 0:1]
    q_miss = q[:, 1:2]
    ones = jnp.full((64, 1), 1.0, jnp.float32)
    u_v = jnp.dot(u * oh_vf, ones, precision=lax.Precision.HIGHEST,
                  preferred_element_type=jnp.float32)

    u_top = jnp.max(jnp.where(oh_v, -1.0, u), axis=1, keepdims=True)
    c1 = jnp.min(jnp.where((u == u_top) & (~oh_v), iota_c, 64),
                 axis=1, keepdims=True)

    g_v = -jnp.log(-jnp.log(u_v + 1e-30) + 1e-30)
    g_1 = -jnp.log(-jnp.log(u_top + 1e-30) + 1e-30)
    s_v = g_v + q_hit
    s_1 = g_1 + q_miss

    widx = jnp.where(s_v > s_1, vi,
                     jnp.where(s_1 > s_v, c1, jnp.minimum(vi, c1)))

    eq_w = iota_c == widx
    vp_ref[...] = eq_w.astype(jnp.float32)
    lnvt_ref[...] = jnp.where(eq_w, pos, neg30)
    lv0_ref[...] = jnp.where(oh_v, pos, neg30)


def kernel(v, time_step, batch, u, log_alphas_bar, log_1_min_alphas_bar):
    N, C = u.shape
    R = 2048
    G = N // R
    vi = v.reshape(N, 1)
    bi = batch.reshape(N, 1)
    tsi = time_step.reshape(-1, 1)
    T = log_alphas_bar.shape[0]
    coef = jnp.zeros((128, 2), jnp.float32)
    coef = coef.at[:T, 0].set(log_alphas_bar).at[:T, 1].set(log_1_min_alphas_bar)

    row_spec = pl.BlockSpec((R, C), lambda i: (i, 0))
    col_spec = pl.BlockSpec((R, 1), lambda i: (i, 0))
    ts_spec = pl.BlockSpec((64, 1), lambda i: (0, 0))
    coef_spec = pl.BlockSpec((128, 2), lambda i: (0, 0))
    out_sds = jax.ShapeDtypeStruct((N, C), jnp.float32)

    vp, lnvt, lv0 = pl.pallas_call(
        _body,
        grid=(G,),
        in_specs=[row_spec, col_spec, col_spec, ts_spec, coef_spec],
        out_specs=[row_spec, row_spec, row_spec],
        out_shape=[out_sds, out_sds, out_sds],
    )(u, vi, bi, tsi, coef)
    return (vp, lnvt, lv0)
